# baseline (speedup 1.0000x reference)
import functools

import jax
import jax.numpy as jnp
from jax import lax
from jax.experimental import pallas as pl
from jax.experimental.pallas import tpu as pltpu

_NUM_CLASSES = 8
_THRESHOLD = 0.5
_NEGPOS_RATIO = 3
_VAR0 = 0.1
_VAR1 = 0.2
_LANES = 128
_IPS = 16


def _sl1(d):
    ad = jnp.abs(d)
    return jnp.where(ad < 1.0, 0.5 * d * d, ad - 0.5)


def _images(tss, locs, confs, pri_env, n_obj, n_priors, rows, n_cls):
    S = len(tss)
    R = rows
    (pcx, pcy, pw, ph, px1, py1, px2, py2, parea, flat) = pri_env

    lx = [locs[s][0 * R:1 * R] for s in range(S)]
    ly = [locs[s][1 * R:2 * R] for s in range(S)]
    lw = [locs[s][2 * R:3 * R] for s in range(S)]
    lh = [locs[s][3 * R:4 * R] for s in range(S)]

    tareas = [[(t[2] - t[0]) * (t[3] - t[1]) for t in tss[s]] for s in range(S)]

    bto = [None] * S
    bti = [None] * S
    bpi = [[] for _ in range(S)]
    BIG = jnp.int32(1 << 30)
    for j in range(n_obj):
        for s in range(S):
            ax1, ay1, ax2, ay2, _ = tss[s][j]
            iw = jnp.maximum(jnp.minimum(ax2, px2) - jnp.maximum(ax1, px1), 0.0)
            ih = jnp.maximum(jnp.minimum(ay2, py2) - jnp.maximum(ay1, py1), 0.0)
            inter = iw * ih
            iou = inter / (tareas[s][j] + parea - inter)
            m = jnp.max(iou)
            bpi[s].append(jnp.min(jnp.where(iou == m, flat, BIG)))
            if j == 0:
                bto[s] = iou
                bti[s] = jnp.zeros((R, _LANES), jnp.int32)
            else:
                bti[s] = jnp.where(iou > bto[s], j, bti[s])
                bto[s] = jnp.maximum(bto[s], iou)

    for j in range(n_obj):
        for s in range(S):
            hit = flat == bpi[s][j]
            bto[s] = jnp.where(hit, 2.0, bto[s])
            bti[s] = jnp.where(hit, j, bti[s])

    pos = [bto[s] >= _THRESHOLD for s in range(S)]
    num_pos = [jnp.sum(pos[s].astype(jnp.int32)) for s in range(S)]

    mx1 = [jnp.full((R, _LANES), tss[s][0][0]) for s in range(S)]
    my1 = [jnp.full((R, _LANES), tss[s][0][1]) for s in range(S)]
    mx2 = [jnp.full((R, _LANES), tss[s][0][2]) for s in range(S)]
    my2 = [jnp.full((R, _LANES), tss[s][0][3]) for s in range(S)]
    mlab = [jnp.full((R, _LANES), tss[s][0][4]) for s in range(S)]
    for j in range(1, n_obj):
        for s in range(S):
            sel = bti[s] == j
            mx1[s] = jnp.where(sel, tss[s][j][0], mx1[s])
            my1[s] = jnp.where(sel, tss[s][j][1], my1[s])
            mx2[s] = jnp.where(sel, tss[s][j][2], mx2[s])
            my2[s] = jnp.where(sel, tss[s][j][3], my2[s])
            mlab[s] = jnp.where(sel, tss[s][j][4], mlab[s])

    ll = []
    for s in range(S):
        ecx = ((mx1[s] + mx2[s]) / 2.0 - pcx) / (_VAR0 * pw)
        ecy = ((my1[s] + my2[s]) / 2.0 - pcy) / (_VAR0 * ph)
        ew = jnp.log((mx2[s] - mx1[s]) / pw) / _VAR1
        eh = jnp.log((my2[s] - my1[s]) / ph) / _VAR1
        ll.append(jnp.sum(jnp.where(
            pos[s],
            _sl1(lx[s] - ecx) + _sl1(ly[s] - ecy) + _sl1(lw[s] - ew)
            + _sl1(lh[s] - eh), 0.0)))

    dx1, dy1, dx2, dy2, darea = [], [], [], [], []
    for s in range(S):
        dcx = pcx + lx[s] * _VAR0 * pw
        dcy = pcy + ly[s] * _VAR0 * ph
        dw = pw * jnp.exp(lw[s] * _VAR1)
        dh = ph * jnp.exp(lh[s] * _VAR1)
        dx1.append(dcx - dw / 2.0)
        dy1.append(dcy - dh / 2.0)
        dx2.append(dcx + dw / 2.0)
        dy2.append(dcy + dh / 2.0)
        darea.append((dx2[s] - dx1[s]) * (dy2[s] - dy1[s]))

    bti1 = [jnp.zeros((R, _LANES), jnp.int32) for _ in range(S)]
    bo1 = [None] * S
    for j in range(n_obj):
        for s in range(S):
            ax1, ay1, ax2, ay2, _ = tss[s][j]
            iw = jnp.maximum(jnp.minimum(ax2, dx2[s]) - jnp.maximum(ax1, dx1[s]), 0.0)
            ih = jnp.maximum(jnp.minimum(ay2, dy2[s]) - jnp.maximum(ay1, dy1[s]), 0.0)
            inter = iw * ih
            iou = inter / (tareas[s][j] + darea[s] - inter)
            if j == 0:
                bo1[s] = iou
            else:
                bti1[s] = jnp.where(iou > bo1[s], j, bti1[s])
                bo1[s] = jnp.maximum(bo1[s], iou)

    gx1 = [jnp.full((R, _LANES), tss[s][0][0]) for s in range(S)]
    gy1 = [jnp.full((R, _LANES), tss[s][0][1]) for s in range(S)]
    gx2 = [jnp.full((R, _LANES), tss[s][0][2]) for s in range(S)]
    gy2 = [jnp.full((R, _LANES), tss[s][0][3]) for s in range(S)]
    for j in range(1, n_obj):
        for s in range(S):
            sel = bti1[s] == j
            gx1[s] = jnp.where(sel, tss[s][j][0], gx1[s])
            gy1[s] = jnp.where(sel, tss[s][j][1], gy1[s])
            gx2[s] = jnp.where(sel, tss[s][j][2], gx2[s])
            gy2[s] = jnp.where(sel, tss[s][j][3], gy2[s])

    lr = []
    for s in range(S):
        riw = jnp.maximum(jnp.minimum(gx2[s], dx2[s]) - jnp.maximum(gx1[s], dx1[s]), 0.0)
        rih = jnp.maximum(jnp.minimum(gy2[s], dy2[s]) - jnp.maximum(gy1[s], dy1[s]), 0.0)
        iog = riw * rih / ((gx2[s] - gx1[s]) * (gy2[s] - gy1[s]))
        arg = jnp.maximum(1.0 - iog, 0.0) + 1e-10
        lr.append(jnp.sum(jnp.where(pos[s], -jnp.log(arg), 0.0)))

    ce, mce, bits, kk = [], [], [], []
    for s in range(S):
        cls = [confs[s][c * R:(c + 1) * R] for c in range(n_cls)]
        cmax = cls[0]
        for c in range(1, n_cls):
            cmax = jnp.maximum(cmax, cls[c])
        ssum = jnp.exp(cls[0] - cmax)
        for c in range(1, n_cls):
            ssum = ssum + jnp.exp(cls[c] - cmax)
        lse = jnp.log(ssum) + cmax
        ct = jnp.where(pos[s], mlab[s].astype(jnp.int32) + 1, 0)
        chosen = jnp.where(ct == 0, cls[0], 0.0)
        for c in range(1, n_cls):
            chosen = chosen + jnp.where(ct == c, cls[c], 0.0)
        ce.append(lse - chosen)
        mce.append(jnp.where(pos[s], 0.0, ce[s]))
        bits.append(lax.bitcast_convert_type(mce[s], jnp.int32))
        kk.append(jnp.minimum(_NEGPOS_RATIO * num_pos[s], n_priors - 1))

    def bs_body(_, carry):
        los, his = carry
        nlo, nhi = [], []
        for s in range(S):
            mid = los[s] + ((his[s] - los[s] + 1) >> 1)
            cnt = jnp.sum((bits[s] >= mid).astype(jnp.int32))
            take = cnt >= kk[s]
            nlo.append(jnp.where(take, mid, los[s]))
            nhi.append(jnp.where(take, his[s], mid - 1))
        return (tuple(nlo), tuple(nhi))

    init = (tuple(jnp.int32(0) for _ in range(S)),
            tuple(jnp.int32(0x7f800000) for _ in range(S)))
    los, _ = lax.fori_loop(0, 31, bs_body, init)

    lc = []
    for s in range(S):
        kth = lax.bitcast_convert_type(los[s], jnp.float32)
        gt = bits[s] > los[s]
        cnt_gt = jnp.sum(gt.astype(jnp.int32))
        lc.append(jnp.sum(jnp.where(pos[s], ce[s], 0.0))
                  + jnp.sum(jnp.where(gt, mce[s], 0.0))
                  + (kk[s] - cnt_gt).astype(jnp.float32) * kth)

    ll_t = ll[0]
    lr_t = lr[0]
    lc_t = lc[0]
    np_t = num_pos[0]
    for s in range(1, S):
        ll_t += ll[s]
        lr_t += lr[s]
        lc_t += lc[s]
        np_t += num_pos[s]
    return ll_t, lr_t, lc_t, np_t.astype(jnp.float32)


def _mbl_body(tgt_ref, loc_ref, conf_ref, pri_ref, out_ref, *, n_obj, n_priors,
              rows, n_cls, ips):
    R = rows
    step = pl.program_id(0)

    row_i = lax.broadcasted_iota(jnp.int32, (R, _LANES), 0)
    col_i = lax.broadcasted_iota(jnp.int32, (R, _LANES), 1)
    flat = row_i * _LANES + col_i

    pri = pri_ref[...]
    pcx = pri[0 * R:1 * R]
    pcy = pri[1 * R:2 * R]
    pw = pri[2 * R:3 * R]
    ph = pri[3 * R:4 * R]
    px1 = pcx - pw / 2.0
    py1 = pcy - ph / 2.0
    px2 = pcx + pw / 2.0
    py2 = pcy + ph / 2.0
    parea = (px2 - px1) * (py2 - py1)
    pri_env = (pcx, pcy, pw, ph, px1, py1, px2, py2, parea, flat)

    tss = []
    locs = []
    confs = []
    for s in range(ips):
        img = step * ips + s
        tss.append([[tgt_ref[img, f, j] for f in range(5)] for j in range(n_obj)])
        locs.append(loc_ref[s])
        confs.append(conf_ref[s])

    ll, lr, lc, npos = _images(tss, locs, confs, pri_env, n_obj, n_priors, R,
                               n_cls)

    r8 = lax.broadcasted_iota(jnp.int32, (8, _LANES), 0)
    acc = (jnp.where(r8 == 0, ll, 0.0) + jnp.where(r8 == 1, lr, 0.0)
           + jnp.where(r8 == 2, lc, 0.0) + jnp.where(r8 == 3, npos, 0.0))

    @pl.when(step == 0)
    def _():
        out_ref[...] = jnp.zeros_like(out_ref)

    out_ref[...] += acc


def kernel(loc_data, conf_data, priors, targets):
    B, P, _ = loc_data.shape
    C = conf_data.shape[-1]
    NOBJ = targets.shape[1]
    R = -(-P // _LANES)
    R = -(-R // 8) * 8
    PP = R * _LANES

    locT = jnp.transpose(loc_data, (0, 2, 1))
    locT = jnp.pad(locT, ((0, 0), (0, 0), (0, PP - P))).reshape(B, 4 * R, _LANES)
    confT = jnp.transpose(conf_data, (0, 2, 1))
    pad_cls = jnp.concatenate(
        [jnp.zeros((1, 1), jnp.float32), jnp.full((C - 1, 1), -30.0)], axis=0)
    confT = jnp.concatenate(
        [confT, jnp.broadcast_to(pad_cls, (B, C, PP - P))], axis=2)
    confT = confT.reshape(B, C * R, _LANES)
    priT = jnp.transpose(priors, (1, 0))
    pad_col = jnp.array([10.5, 10.5, 1.0, 1.0], jnp.float32)[:, None]
    priT = jnp.concatenate(
        [priT, jnp.broadcast_to(pad_col, (4, PP - P))], axis=1).reshape(4 * R, _LANES)
    tgtT = jnp.transpose(targets, (0, 2, 1))

    out = pl.pallas_call(
        functools.partial(_mbl_body, n_obj=NOBJ, n_priors=P, rows=R, n_cls=C,
                          ips=_IPS),
        grid=(B // _IPS,),
        in_specs=[
            pl.BlockSpec(memory_space=pltpu.SMEM),
            pl.BlockSpec((_IPS, 4 * R, _LANES), lambda i: (i, 0, 0)),
            pl.BlockSpec((_IPS, C * R, _LANES), lambda i: (i, 0, 0)),
            pl.BlockSpec((4 * R, _LANES), lambda i: (0, 0)),
        ],
        out_specs=pl.BlockSpec((8, _LANES), lambda i: (0, 0)),
        out_shape=jax.ShapeDtypeStruct((8, _LANES), jnp.float32),
    )(tgtT, locT, confT, priT)

    n = out[3, 0]
    return (out[0, 0] / n, out[1, 0] / n, out[2, 0] / n)


# speedup vs baseline: 1.7549x; 1.7549x over previous
import functools

import jax
import jax.numpy as jnp
from jax import lax
from jax.experimental import pallas as pl
from jax.experimental.pallas import tpu as pltpu

_NUM_CLASSES = 8
_THRESHOLD = 0.5
_NEGPOS_RATIO = 3
_VAR0 = 0.1
_VAR1 = 0.2
_LANES = 128
_IPS = 8


def _sl1(d):
    ad = jnp.abs(d)
    return jnp.where(ad < 1.0, 0.5 * d * d, ad - 0.5)


def _lane_sum(x, ones_col):
    return lax.dot_general(x, ones_col, (((1,), (0,)), ((), ())),
                           precision=lax.Precision.HIGHEST,
                           preferred_element_type=jnp.float32)


def _full_sum(x, ones_col):
    return _lane_sum(jnp.sum(x, axis=0, keepdims=True), ones_col)


def _images(tss, locs, confs, pri_env, n_obj, n_priors, rows, n_cls):
    S = len(tss)
    R = rows
    (pcx, pcy, pw, ph, px1, py1, px2, py2, parea, flat, row_i, ones_col) = pri_env

    lx = [locs[s][0 * R:1 * R] for s in range(S)]
    ly = [locs[s][1 * R:2 * R] for s in range(S)]
    lw = [locs[s][2 * R:3 * R] for s in range(S)]
    lh = [locs[s][3 * R:4 * R] for s in range(S)]

    tareas = [[(t[2] - t[0]) * (t[3] - t[1]) for t in tss[s]] for s in range(S)]

    bto = [None] * S
    bti = [None] * S
    cms = [[] for _ in range(S)]
    crs = [[] for _ in range(S)]
    BIG = jnp.int32(1 << 30)
    for j in range(n_obj):
        for s in range(S):
            ax1, ay1, ax2, ay2, _ = tss[s][j]
            iw = jnp.maximum(jnp.minimum(ax2, px2) - jnp.maximum(ax1, px1), 0.0)
            ih = jnp.maximum(jnp.minimum(ay2, py2) - jnp.maximum(ay1, py1), 0.0)
            inter = iw * ih
            iou = inter / (tareas[s][j] + parea - inter)
            cm = jnp.max(iou, axis=0, keepdims=True)
            cms[s].append(cm)
            crs[s].append(jnp.min(jnp.where(iou == cm, row_i, BIG), axis=0,
                                  keepdims=True))
            if j == 0:
                bto[s] = iou
                bti[s] = jnp.zeros((R, _LANES), jnp.int32)
            else:
                bti[s] = jnp.where(iou > bto[s], j, bti[s])
                bto[s] = jnp.maximum(bto[s], iou)

    n_pad = -(-n_obj // 8) * 8
    lane_np = lax.broadcasted_iota(jnp.int32, (n_pad, _LANES), 1)
    bpi = []
    for s in range(S):
        pad_m = ([jnp.full((n_pad - n_obj, _LANES), -1.0)]
                 if n_pad > n_obj else [])
        pad_r = ([jnp.zeros((n_pad - n_obj, _LANES), jnp.int32)]
                 if n_pad > n_obj else [])
        CM = jnp.concatenate(cms[s] + pad_m, axis=0)
        CR = jnp.concatenate(crs[s] + pad_r, axis=0)
        m_row = jnp.max(CM, axis=1, keepdims=True)
        cstar = jnp.min(jnp.where(CM == m_row, lane_np, BIG), axis=1,
                        keepdims=True)
        rstar = jnp.max(jnp.where(lane_np == cstar, CR, 0), axis=1,
                        keepdims=True)
        bpi.append(rstar * _LANES + cstar)

    for j in range(n_obj):
        for s in range(S):
            hit = flat == bpi[s][j:j + 1, 0:1]
            bto[s] = jnp.where(hit, 2.0, bto[s])
            bti[s] = jnp.where(hit, j, bti[s])

    pos = [bto[s] >= _THRESHOLD for s in range(S)]
    num_pos = [_full_sum(pos[s].astype(jnp.float32), ones_col)
               for s in range(S)]

    mx1 = [jnp.full((R, _LANES), tss[s][0][0]) for s in range(S)]
    my1 = [jnp.full((R, _LANES), tss[s][0][1]) for s in range(S)]
    mx2 = [jnp.full((R, _LANES), tss[s][0][2]) for s in range(S)]
    my2 = [jnp.full((R, _LANES), tss[s][0][3]) for s in range(S)]
    mlab = [jnp.full((R, _LANES), tss[s][0][4]) for s in range(S)]
    for j in range(1, n_obj):
        for s in range(S):
            sel = bti[s] == j
            mx1[s] = jnp.where(sel, tss[s][j][0], mx1[s])
            my1[s] = jnp.where(sel, tss[s][j][1], my1[s])
            mx2[s] = jnp.where(sel, tss[s][j][2], mx2[s])
            my2[s] = jnp.where(sel, tss[s][j][3], my2[s])
            mlab[s] = jnp.where(sel, tss[s][j][4], mlab[s])

    ll = []
    for s in range(S):
        ecx = ((mx1[s] + mx2[s]) / 2.0 - pcx) / (_VAR0 * pw)
        ecy = ((my1[s] + my2[s]) / 2.0 - pcy) / (_VAR0 * ph)
        ew = jnp.log((mx2[s] - mx1[s]) / pw) / _VAR1
        eh = jnp.log((my2[s] - my1[s]) / ph) / _VAR1
        ll.append(_full_sum(jnp.where(
            pos[s],
            _sl1(lx[s] - ecx) + _sl1(ly[s] - ecy) + _sl1(lw[s] - ew)
            + _sl1(lh[s] - eh), 0.0), ones_col))

    dx1, dy1, dx2, dy2, darea = [], [], [], [], []
    for s in range(S):
        dcx = pcx + lx[s] * _VAR0 * pw
        dcy = pcy + ly[s] * _VAR0 * ph
        dw = pw * jnp.exp(lw[s] * _VAR1)
        dh = ph * jnp.exp(lh[s] * _VAR1)
        dx1.append(dcx - dw / 2.0)
        dy1.append(dcy - dh / 2.0)
        dx2.append(dcx + dw / 2.0)
        dy2.append(dcy + dh / 2.0)
        darea.append((dx2[s] - dx1[s]) * (dy2[s] - dy1[s]))

    bti1 = [jnp.zeros((R, _LANES), jnp.int32) for _ in range(S)]
    bo1 = [None] * S
    for j in range(n_obj):
        for s in range(S):
            ax1, ay1, ax2, ay2, _ = tss[s][j]
            iw = jnp.maximum(jnp.minimum(ax2, dx2[s]) - jnp.maximum(ax1, dx1[s]), 0.0)
            ih = jnp.maximum(jnp.minimum(ay2, dy2[s]) - jnp.maximum(ay1, dy1[s]), 0.0)
            inter = iw * ih
            iou = inter / (tareas[s][j] + darea[s] - inter)
            if j == 0:
                bo1[s] = iou
            else:
                bti1[s] = jnp.where(iou > bo1[s], j, bti1[s])
                bo1[s] = jnp.maximum(bo1[s], iou)

    gx1 = [jnp.full((R, _LANES), tss[s][0][0]) for s in range(S)]
    gy1 = [jnp.full((R, _LANES), tss[s][0][1]) for s in range(S)]
    gx2 = [jnp.full((R, _LANES), tss[s][0][2]) for s in range(S)]
    gy2 = [jnp.full((R, _LANES), tss[s][0][3]) for s in range(S)]
    for j in range(1, n_obj):
        for s in range(S):
            sel = bti1[s] == j
            gx1[s] = jnp.where(sel, tss[s][j][0], gx1[s])
            gy1[s] = jnp.where(sel, tss[s][j][1], gy1[s])
            gx2[s] = jnp.where(sel, tss[s][j][2], gx2[s])
            gy2[s] = jnp.where(sel, tss[s][j][3], gy2[s])

    lr = []
    for s in range(S):
        riw = jnp.maximum(jnp.minimum(gx2[s], dx2[s]) - jnp.maximum(gx1[s], dx1[s]), 0.0)
        rih = jnp.maximum(jnp.minimum(gy2[s], dy2[s]) - jnp.maximum(gy1[s], dy1[s]), 0.0)
        iog = riw * rih / ((gx2[s] - gx1[s]) * (gy2[s] - gy1[s]))
        arg = jnp.maximum(1.0 - iog, 0.0) + 1e-10
        lr.append(_full_sum(jnp.where(pos[s], -jnp.log(arg), 0.0), ones_col))

    ce, mce, bits, kk = [], [], [], []
    for s in range(S):
        cls = [confs[s][c * R:(c + 1) * R] for c in range(n_cls)]
        cmax = cls[0]
        for c in range(1, n_cls):
            cmax = jnp.maximum(cmax, cls[c])
        ssum = jnp.exp(cls[0] - cmax)
        for c in range(1, n_cls):
            ssum = ssum + jnp.exp(cls[c] - cmax)
        lse = jnp.log(ssum) + cmax
        ct = jnp.where(pos[s], mlab[s].astype(jnp.int32) + 1, 0)
        chosen = jnp.where(ct == 0, cls[0], 0.0)
        for c in range(1, n_cls):
            chosen = chosen + jnp.where(ct == c, cls[c], 0.0)
        ce.append(lse - chosen)
        mce.append(jnp.where(pos[s], 0.0, ce[s]))
        bits.append(lax.bitcast_convert_type(mce[s], jnp.int32))
        kk.append(jnp.minimum(_NEGPOS_RATIO * num_pos[s],
                              jnp.float32(n_priors - 1)))

    def bs_body(_, carry):
        los, his = carry
        nlo, nhi = [], []
        for s in range(S):
            mid = los[s] + ((his[s] - los[s] + 1) >> 1)
            cnt = _full_sum((bits[s] >= mid).astype(jnp.float32), ones_col)
            take = cnt >= kk[s]
            nlo.append(jnp.where(take, mid, los[s]))
            nhi.append(jnp.where(take, his[s], mid - 1))
        return (tuple(nlo), tuple(nhi))

    init = (tuple(jnp.zeros((1, 1), jnp.int32) for _ in range(S)),
            tuple(jnp.full((1, 1), 0x7f800000, jnp.int32) for _ in range(S)))
    los, _ = lax.fori_loop(0, 31, bs_body, init)

    lc = []
    for s in range(S):
        kth = lax.bitcast_convert_type(los[s], jnp.float32)
        gt = bits[s] > los[s]
        cnt_gt = _full_sum(gt.astype(jnp.float32), ones_col)
        lc.append(_full_sum(jnp.where(pos[s], ce[s], 0.0), ones_col)
                  + _full_sum(jnp.where(gt, mce[s], 0.0), ones_col)
                  + (kk[s] - cnt_gt) * kth)

    ll_t = ll[0]
    lr_t = lr[0]
    lc_t = lc[0]
    np_t = num_pos[0]
    for s in range(1, S):
        ll_t += ll[s]
        lr_t += lr[s]
        lc_t += lc[s]
        np_t += num_pos[s]
    return ll_t, lr_t, lc_t, np_t


def _mbl_body(tgt_ref, loc_ref, conf_ref, pri_ref, out_ref, *, n_obj, n_priors,
              rows, n_cls, ips):
    R = rows
    step = pl.program_id(0)

    row_i = lax.broadcasted_iota(jnp.int32, (R, _LANES), 0)
    col_i = lax.broadcasted_iota(jnp.int32, (R, _LANES), 1)
    flat = row_i * _LANES + col_i

    pri = pri_ref[...]
    pcx = pri[0 * R:1 * R]
    pcy = pri[1 * R:2 * R]
    pw = pri[2 * R:3 * R]
    ph = pri[3 * R:4 * R]
    px1 = pcx - pw / 2.0
    py1 = pcy - ph / 2.0
    px2 = pcx + pw / 2.0
    py2 = pcy + ph / 2.0
    parea = (px2 - px1) * (py2 - py1)
    ones_col = jnp.ones((_LANES, 1), jnp.float32)
    pri_env = (pcx, pcy, pw, ph, px1, py1, px2, py2, parea, flat, row_i,
               ones_col)

    tss = []
    locs = []
    confs = []
    for s in range(ips):
        img = step * ips + s
        tss.append([[tgt_ref[img, f, j] for f in range(5)] for j in range(n_obj)])
        locs.append(loc_ref[s])
        confs.append(conf_ref[s])

    ll, lr, lc, npos = _images(tss, locs, confs, pri_env, n_obj, n_priors, R,
                               n_cls)

    r8 = lax.broadcasted_iota(jnp.int32, (8, _LANES), 0)
    acc = (jnp.where(r8 == 0, jnp.broadcast_to(ll, (8, _LANES)), 0.0)
           + jnp.where(r8 == 1, jnp.broadcast_to(lr, (8, _LANES)), 0.0)
           + jnp.where(r8 == 2, jnp.broadcast_to(lc, (8, _LANES)), 0.0)
           + jnp.where(r8 == 3, jnp.broadcast_to(npos, (8, _LANES)), 0.0))

    @pl.when(step == 0)
    def _():
        out_ref[...] = jnp.zeros_like(out_ref)

    out_ref[...] += acc


def kernel(loc_data, conf_data, priors, targets):
    B, P, _ = loc_data.shape
    C = conf_data.shape[-1]
    NOBJ = targets.shape[1]
    R = -(-P // _LANES)
    R = -(-R // 8) * 8
    PP = R * _LANES

    locT = jnp.transpose(loc_data, (0, 2, 1))
    locT = jnp.pad(locT, ((0, 0), (0, 0), (0, PP - P))).reshape(B, 4 * R, _LANES)
    confT = jnp.transpose(conf_data, (0, 2, 1))
    pad_cls = jnp.concatenate(
        [jnp.zeros((1, 1), jnp.float32), jnp.full((C - 1, 1), -30.0)], axis=0)
    confT = jnp.concatenate(
        [confT, jnp.broadcast_to(pad_cls, (B, C, PP - P))], axis=2)
    confT = confT.reshape(B, C * R, _LANES)
    priT = jnp.transpose(priors, (1, 0))
    pad_col = jnp.array([10.5, 10.5, 1.0, 1.0], jnp.float32)[:, None]
    priT = jnp.concatenate(
        [priT, jnp.broadcast_to(pad_col, (4, PP - P))], axis=1).reshape(4 * R, _LANES)
    tgtT = jnp.transpose(targets, (0, 2, 1))

    out = pl.pallas_call(
        functools.partial(_mbl_body, n_obj=NOBJ, n_priors=P, rows=R, n_cls=C,
                          ips=_IPS),
        grid=(B // _IPS,),
        in_specs=[
            pl.BlockSpec(memory_space=pltpu.SMEM),
            pl.BlockSpec((_IPS, 4 * R, _LANES), lambda i: (i, 0, 0)),
            pl.BlockSpec((_IPS, C * R, _LANES), lambda i: (i, 0, 0)),
            pl.BlockSpec((4 * R, _LANES), lambda i: (0, 0)),
        ],
        out_specs=pl.BlockSpec((8, _LANES), lambda i: (0, 0)),
        out_shape=jax.ShapeDtypeStruct((8, _LANES), jnp.float32),
    )(tgtT, locT, confT, priT)

    n = out[3, 0]
    return (out[0, 0] / n, out[1, 0] / n, out[2, 0] / n)
